# baseline (speedup 1.0000x reference)
import functools

import jax
import jax.numpy as jnp
from jax import lax
from jax.experimental import pallas as pl
from jax.experimental.pallas import tpu as pltpu
from jax.experimental.pallas import tpu_sc as plsc

D = 64
C = 4
NC = 2
NS = 16
NW = NC * NS
CHUNK = 128
LANES = 16


def _rtable_tc_kernel(dw_ref, wt_ref, b_ref, out_ref):
    dw = dw_ref[...]
    m = jnp.max(dw)
    e = jnp.exp(dw - m)
    sm = e / jnp.sum(e)
    out_ref[...] = sm[:, None] * wt_ref[...] + b_ref[...][None, :]


def _make_rtable(decay_weights, W, b):
    wt = W.T
    return pl.pallas_call(
        _rtable_tc_kernel,
        out_shape=jax.ShapeDtypeStruct((C, D), jnp.float32),
    )(decay_weights, wt, b)


def _make_sc_gather(n_total):
    per_w = n_total // NW
    n_chunks = per_w // CHUNK
    mesh = plsc.VectorSubcoreMesh(core_axis_name="c", subcore_axis_name="s")

    @functools.partial(
        pl.kernel,
        mesh=mesh,
        out_type=jax.ShapeDtypeStruct((n_total, D), jnp.float32),
        compiler_params=pltpu.CompilerParams(use_tc_tiling_on_sc=False),
        scratch_types=[
            pltpu.VMEM((CHUNK,), jnp.int32),
            pltpu.VMEM((CHUNK,), jnp.int32),
            pltpu.VMEM((CHUNK, D), jnp.float32),
            pltpu.VMEM((CHUNK, D), jnp.float32),
            pltpu.SemaphoreType.DMA,
            pltpu.SemaphoreType.DMA,
        ],
    )
    def sc_kernel(table_hbm, qidx_hbm, ridx_hbm, rtab_hbm, out_hbm,
                  qi_v, ri_v, qr_v, rr_v, qsem, rsem):
        wid = lax.axis_index("s") * NC + lax.axis_index("c")
        base = wid * per_w

        @pl.loop(0, n_chunks)
        def _chunk(g):
            off = base + g * CHUNK
            pltpu.sync_copy(qidx_hbm.at[pl.ds(off, CHUNK)], qi_v)
            pltpu.sync_copy(ridx_hbm.at[pl.ds(off, CHUNK)], ri_v)
            qcp = pltpu.async_copy(table_hbm.at[qi_v], qr_v, qsem)
            rcp = pltpu.async_copy(rtab_hbm.at[ri_v], rr_v, rsem)
            qcp.wait()
            rcp.wait()

            @pl.loop(0, CHUNK)
            def _row(i):
                for c in range(D // LANES):
                    sl = (i, pl.ds(c * LANES, LANES))
                    qr_v.at[*sl][...] = qr_v.at[*sl][...] + rr_v.at[*sl][...]

            pltpu.sync_copy(qr_v, out_hbm.at[pl.ds(off, CHUNK)])

    return sc_kernel


def kernel(q_data, r_data, table, decay_weights, W, b):
    bsz, seq = q_data.shape
    n_total = bsz * seq
    r_table = _make_rtable(decay_weights, W, b)
    q_flat = q_data.reshape(n_total)
    r_flat = r_data.reshape(n_total)
    out = _make_sc_gather(n_total)(table, q_flat, r_flat, r_table)
    return out.reshape(bsz, seq, D)
